# baseline (speedup 1.0000x reference)
import functools

import jax
import jax.numpy as jnp
from jax import lax
from jax.experimental import pallas as pl
from jax.experimental.pallas import tpu as pltpu
from jax.experimental.pallas import tpu_sc as plsc

_NC, _NS, _LANES = 2, 16, 16
_NW = _NC * _NS
_D = 256
_P = 64 * 1024
_PPW = _P // _NW
_CH = 64
_NCH = _PPW // _CH
_IDXROWS = _PPW // _CH
_CHW = _CH * 2 * _D


@functools.partial(
    pl.kernel,
    mesh=plsc.VectorSubcoreMesh(core_axis_name="c", subcore_axis_name="s"),
    out_type=jax.ShapeDtypeStruct((_P * 2 * _D,), jnp.float32),
    compiler_params=pltpu.CompilerParams(use_tc_tiling_on_sc=False,
                                         needs_layout_passes=False),
    scratch_types=[
        pltpu.VMEM((_IDXROWS, _CH), jnp.int32),
        pltpu.VMEM((_IDXROWS, _CH), jnp.int32),
        pltpu.VMEM((2 * 64 * _D,), jnp.float32),
        pltpu.VMEM((_CHW,), jnp.float32),
        pltpu.VMEM((_CHW,), jnp.float32),
        pltpu.SemaphoreType.DMA,
        pltpu.SemaphoreType.DMA,
    ],
)
def _sc_lookup(idx_x_hbm, idx_y_hbm, tbl_hbm, out_hbm,
               idxx_v, idxy_v, tbl_v, buf0, buf1, so0, so1):
    bufs = (buf0, buf1)
    sos = (so0, so1)
    wid = lax.axis_index("s") * _NC + lax.axis_index("c")
    base = wid * _PPW

    pltpu.sync_copy(tbl_hbm, tbl_v)

    pltpu.sync_copy(idx_x_hbm.at[pl.ds(wid * _IDXROWS, _IDXROWS)], idxx_v)
    pltpu.sync_copy(idx_y_hbm.at[pl.ds(wid * _IDXROWS, _IDXROWS)], idxy_v)

    lane = lax.iota(jnp.int32, _LANES)

    def compute(c, b):
        buf = bufs[b]
        for g in range(_CH // _LANES):
            pxb = idxx_v[c, pl.ds(g * _LANES, _LANES)] * _D
            pyb = idxy_v[c, pl.ds(g * _LANES, _LANES)] * _D + 64 * _D
            rb = (lane + g * _LANES) * (2 * _D)
            rb2 = rb + _D

            @plsc.parallel_loop(0, _D, unroll=8,
                                carry=jnp.zeros((_LANES,), jnp.int32))
            def _cols(j, jv):
                xv = plsc.load_gather(tbl_v, [pxb + jv])
                plsc.store_scatter(buf, [rb + jv], xv)
                yv = plsc.load_gather(tbl_v, [pyb + jv])
                plsc.store_scatter(buf, [rb2 + jv], yv)
                return jv + 1

    def start_out(c, b):
        pltpu.async_copy(
            bufs[b], out_hbm.at[pl.ds((base + c * _CH) * 2 * _D, _CHW)],
            sos[b])

    def wait_out(b):
        pltpu.make_async_copy(bufs[b], out_hbm.at[pl.ds(0, _CHW)],
                              sos[b]).wait()

    compute(0, 0)
    start_out(0, 0)
    compute(1, 1)
    start_out(1, 1)

    def pair(s, carry):
        for b in range(2):
            c = 2 * s + b
            wait_out(b)
            compute(c, b)
            start_out(c, b)
        return carry

    lax.fori_loop(1, _NCH // 2, pair, 0)

    wait_out(0)
    wait_out(1)


def kernel(position_inds, col_embed, row_embed):
    pi = position_inds.astype(jnp.int32)
    idx_x = pi[:, :, 0].reshape(_P // _CH, _CH)
    idx_y = pi[:, :, 1].reshape(_P // _CH, _CH)
    tbl = jnp.concatenate([col_embed, row_embed], axis=0).reshape(-1)
    out = _sc_lookup(idx_x, idx_y, tbl)
    return out.reshape(64, 1024, 2 * _D)


# speedup vs baseline: 1.4576x; 1.4576x over previous
import functools

import jax
import jax.numpy as jnp
from jax import lax
from jax.experimental import pallas as pl
from jax.experimental.pallas import tpu as pltpu
from jax.experimental.pallas import tpu_sc as plsc

_NC, _NS, _LANES = 2, 16, 16
_NW = _NC * _NS
_D = 256
_P = 64 * 1024
_PPW = _P // _NW
_CH = 32
_NCH = _PPW // _CH
_IDXROWS = _PPW // _CH
_CHW = _CH * 2 * _D


@functools.partial(
    pl.kernel,
    mesh=plsc.VectorSubcoreMesh(core_axis_name="c", subcore_axis_name="s"),
    out_type=jax.ShapeDtypeStruct((_P * 2 * _D,), jnp.float32),
    compiler_params=pltpu.CompilerParams(use_tc_tiling_on_sc=False,
                                         needs_layout_passes=False),
    scratch_types=[
        pltpu.VMEM((_IDXROWS, _CH), jnp.int32),
        pltpu.VMEM((_IDXROWS, _CH), jnp.int32),
        pltpu.VMEM((2 * 64 * _D,), jnp.float32),
        pltpu.VMEM((_CHW,), jnp.float32),
        pltpu.VMEM((_CHW,), jnp.float32),
        pltpu.SemaphoreType.DMA,
        pltpu.SemaphoreType.DMA,
    ],
)
def _sc_lookup(idx_x_hbm, idx_y_hbm, tbl_hbm, out_hbm,
               idxx_v, idxy_v, tbl_v, buf0, buf1, so0, so1):
    bufs = (buf0, buf1)
    sos = (so0, so1)
    wid = lax.axis_index("s") * _NC + lax.axis_index("c")
    base = wid * _PPW

    pltpu.sync_copy(tbl_hbm, tbl_v)

    pltpu.sync_copy(idx_x_hbm.at[pl.ds(wid * _IDXROWS, _IDXROWS)], idxx_v)
    pltpu.sync_copy(idx_y_hbm.at[pl.ds(wid * _IDXROWS, _IDXROWS)], idxy_v)

    def compute(c, b):
        buf = bufs[b]
        for g in range(_CH // _LANES):
            pxv = idxx_v[c, pl.ds(g * _LANES, _LANES)] * _D
            pyv = idxy_v[c, pl.ds(g * _LANES, _LANES)] * _D + 64 * _D
            for i in range(_LANES):
                bx = pxv[i]
                by = pyv[i]
                rb = (g * _LANES + i) * (2 * _D)
                for k in range(_D // _LANES):
                    buf[pl.ds(rb + _LANES * k, _LANES)] = (
                        tbl_v[pl.ds(bx + _LANES * k, _LANES)])
                for k in range(_D // _LANES):
                    buf[pl.ds(rb + _D + _LANES * k, _LANES)] = (
                        tbl_v[pl.ds(by + _LANES * k, _LANES)])

    def start_out(c, b):
        pltpu.async_copy(
            bufs[b], out_hbm.at[pl.ds((base + c * _CH) * 2 * _D, _CHW)],
            sos[b])

    def wait_out(b):
        pltpu.make_async_copy(bufs[b], out_hbm.at[pl.ds(0, _CHW)],
                              sos[b]).wait()

    def chunk_step(c, carry):
        even = (c & 1) == 0

        @pl.when(even)
        def _even():
            @pl.when(c >= 2)
            def _w():
                wait_out(0)
            compute(c, 0)
            start_out(c, 0)

        @pl.when(jnp.logical_not(even))
        def _odd():
            @pl.when(c >= 2)
            def _w():
                wait_out(1)
            compute(c, 1)
            start_out(c, 1)

        return carry

    lax.fori_loop(0, _NCH, chunk_step, 0)

    wait_out(0)
    wait_out(1)


def kernel(position_inds, col_embed, row_embed):
    pi = position_inds.astype(jnp.int32)
    idx_x = pi[:, :, 0].reshape(_P // _CH, _CH)
    idx_y = pi[:, :, 1].reshape(_P // _CH, _CH)
    tbl = jnp.concatenate([col_embed, row_embed], axis=0).reshape(-1)
    out = _sc_lookup(idx_x, idx_y, tbl)
    return out.reshape(64, 1024, 2 * _D)


# speedup vs baseline: 4.6978x; 3.2230x over previous
import functools

import jax
import jax.numpy as jnp
from jax import lax
from jax.experimental import pallas as pl
from jax.experimental.pallas import tpu as pltpu
from jax.experimental.pallas import tpu_sc as plsc

_NC, _NS, _LANES = 2, 16, 16
_NW = _NC * _NS
_D = 256
_W = 2 * _D
_P = 64 * 1024
_PPW = _P // _NW
_CH = 64
_NCH = _PPW // _CH
_IDXROWS = _PPW // _CH
_XPT = 64 // _NS


@functools.partial(
    pl.kernel,
    mesh=plsc.VectorSubcoreMesh(core_axis_name="c", subcore_axis_name="s"),
    out_type=(
        jax.ShapeDtypeStruct((_P, _W), jnp.float32),
        jax.ShapeDtypeStruct((64 * 64, _W), jnp.float32),
        jax.ShapeDtypeStruct((64 * 64, _W), jnp.float32),
    ),
    scratch_types=[
        pltpu.VMEM((_IDXROWS, _CH), jnp.int32),
        pltpu.VMEM((_IDXROWS, _CH), jnp.int32),
        pltpu.VMEM((_XPT, _D), jnp.float32),
        pltpu.VMEM((64, _D), jnp.float32),
        pltpu.VMEM((64, _W), jnp.float32),
        pltpu.VMEM((_CH, _W), jnp.float32),
        pltpu.VMEM((_CH, _W), jnp.float32),
        pltpu.SemaphoreType.DMA,
        pltpu.SemaphoreType.DMA,
        pltpu.SemaphoreType.DMA,
        pltpu.SemaphoreType.DMA,
    ],
)
def _sc_lookup(idx_x_hbm, idx_y_hbm, col_hbm, row_hbm,
               out_hbm, cross0_hbm, cross1_hbm,
               idxx_v, idxy_v, colblk_v, rowtbl_v, blk_v, buf0, buf1,
               sg0, sg1, so0, so1):
    bufs = (buf0, buf1)
    sgs = (sg0, sg1)
    sos = (so0, so1)
    cid = lax.axis_index("c")
    sid = lax.axis_index("s")
    wid = sid * _NC + cid
    base = wid * _PPW

    pltpu.sync_copy(col_hbm.at[pl.ds(sid * _XPT, _XPT)], colblk_v)
    pltpu.sync_copy(row_hbm, rowtbl_v)

    def build_into(cross_hbm):
        for xi in range(_XPT):
            left = [colblk_v[xi, pl.ds(k * _LANES, _LANES)]
                    for k in range(_D // _LANES)]

            def fill_row(r, carry):
                for k in range(_D // _LANES):
                    blk_v[r, pl.ds(k * _LANES, _LANES)] = left[k]
                for k in range(_D // _LANES):
                    blk_v[r, pl.ds(_D + k * _LANES, _LANES)] = (
                        rowtbl_v[r, pl.ds(k * _LANES, _LANES)])
                return carry

            lax.fori_loop(0, 64, fill_row, 0)
            pltpu.sync_copy(
                blk_v, cross_hbm.at[pl.ds((sid * _XPT + xi) * 64, 64)])

    @pl.when(cid == 0)
    def _b0():
        build_into(cross0_hbm)

    @pl.when(cid == 1)
    def _b1():
        build_into(cross1_hbm)

    pltpu.sync_copy(idx_x_hbm.at[pl.ds(wid * _IDXROWS, _IDXROWS)], idxx_v)
    pltpu.sync_copy(idx_y_hbm.at[pl.ds(wid * _IDXROWS, _IDXROWS)], idxy_v)

    def comb_row(i, carry):
        def comb_vec(j, c2):
            sl = pl.ds(j * _LANES, _LANES)
            idxx_v[i, sl] = idxx_v[i, sl] * 64 + idxy_v[i, sl]
            return c2
        return lax.fori_loop(0, _CH // _LANES, comb_vec, carry)

    lax.fori_loop(0, _IDXROWS, comb_row, 0)

    plsc.subcore_barrier()

    def gather_phase(cross_hbm):
        def start_gather(c, b):
            pltpu.async_copy(cross_hbm.at[idxx_v.at[c]], bufs[b], sgs[b])

        def wait_gather(b):
            pltpu.make_async_copy(
                cross_hbm.at[idxx_v.at[0]], bufs[b], sgs[b]).wait()

        def out_desc(c, b):
            return pltpu.make_async_copy(
                bufs[b], out_hbm.at[pl.ds(base + c * _CH, _CH)], sos[b])

        start_gather(0, 0)
        start_gather(1, 1)

        def pair(s, carry):
            for b in range(2):
                c = 2 * s + b
                wait_gather(b)
                out_desc(c, b).start()
                out_desc(c, b).wait()
                start_gather(c + 2, b)
            return carry

        lax.fori_loop(0, _NCH // 2 - 1, pair, 0)

        for b in range(2):
            c = _NCH - 2 + b
            wait_gather(b)
            out_desc(c, b).start()
            out_desc(c, b).wait()

    @pl.when(cid == 0)
    def _g0():
        gather_phase(cross0_hbm)

    @pl.when(cid == 1)
    def _g1():
        gather_phase(cross1_hbm)


def kernel(position_inds, col_embed, row_embed):
    pi = position_inds.astype(jnp.int32)
    idx_x = pi[:, :, 0].reshape(_P // _CH, _CH)
    idx_y = pi[:, :, 1].reshape(_P // _CH, _CH)
    out, _, _ = _sc_lookup(idx_x, idx_y, col_embed, row_embed)
    return out.reshape(64, 1024, _W)
